# baseline (speedup 1.0000x reference)
import functools

import jax
import jax.numpy as jnp
from jax.experimental import pallas as pl
from jax.experimental.pallas import tpu as pltpu

_B, _T, _N = 4, 8, 1024
_DIN, _DH, _DOUT = 256, 256, 256
_HID = _DH // 4
_BT = _B * _T
_BF = jnp.bfloat16


def _dhg_body(feats_ref, gs_ref, wcat_ref, bcat_ref, att_W_ref, att_a_ref,
              att_h_ref, fc_Wt_ref, fc_b_ref, out_ref, sc_ref):
    x = feats_ref[0]
    h = jnp.dot(x, wcat_ref[...], preferred_element_type=jnp.float32)
    h = h + bcat_ref[...]
    hb = h.astype(_BF)

    a0 = att_a_ref[0:_HID, :]
    a1 = att_a_ref[_HID:2 * _HID, :]
    v2 = jnp.dot(att_W_ref[...], a0,
                 preferred_element_type=jnp.float32)
    u = jnp.dot(att_W_ref[...], a1,
                preferred_element_type=jnp.float32)
    c = jnp.dot(att_h_ref[...], u,
                preferred_element_type=jnp.float32)

    branches = []
    logits = []
    for i in range(4):
        ai = jnp.dot(gs_ref[i], hb[:, i * _DH:(i + 1) * _DH],
                     preferred_element_type=jnp.float32)
        ai = jnp.maximum(ai, 0.0)
        branches.append(ai)
        ei = jnp.dot(ai, v2, preferred_element_type=jnp.float32) + c
        logits.append(ei)

    e = jnp.concatenate(logits, axis=1)
    e = jnp.where(e >= 0.0, e, 0.01 * e)
    m = jnp.max(e, axis=1, keepdims=True)
    ex = jnp.exp(e - m)
    s = ex / jnp.sum(ex, axis=1, keepdims=True)
    sc_ref[0] = s

    acc = s[:, 0:1] * branches[0]
    for i in range(1, 4):
        acc = acc + s[:, i:i + 1] * branches[i]
    acc = jnp.maximum(acc, 0.0)

    y = jnp.dot(acc.astype(_BF), fc_Wt_ref[...],
                preferred_element_type=jnp.float32) + fc_b_ref[...]
    out_ref[0] = jnp.maximum(y, 0.0)


@jax.jit
def kernel(feats, G0, G1, G2, G3, W0, b0, W1, b1, W2, b2, W3, b3,
           att_W, att_h, att_a, fc_W, fc_b):
    gs = jnp.stack([G1, G0, G3, G2]).astype(_BF)
    wcat = jnp.concatenate([W1, W0, W3, W2], axis=1).astype(_BF)
    bcat = jnp.concatenate([b1, b0, b3, b2]).reshape(1, 4 * _DH)
    x = feats.reshape(_BT, _N, _DIN).astype(_BF)
    fc_Wt = fc_W.T.astype(_BF)
    att_h_row = att_h.reshape(1, _DH)
    fc_b_row = fc_b.reshape(1, _DOUT)

    y, s = pl.pallas_call(
        _dhg_body,
        grid=(_BT,),
        in_specs=[
            pl.BlockSpec((1, _N, _DIN), lambda i: (i, 0, 0)),
            pl.BlockSpec((4, _N, _N), lambda i: (0, 0, 0)),
            pl.BlockSpec((_DIN, 4 * _DH), lambda i: (0, 0)),
            pl.BlockSpec((1, 4 * _DH), lambda i: (0, 0)),
            pl.BlockSpec((_DH, _HID), lambda i: (0, 0)),
            pl.BlockSpec((2 * _HID, 1), lambda i: (0, 0)),
            pl.BlockSpec((1, _DH), lambda i: (0, 0)),
            pl.BlockSpec((_DH, _DOUT), lambda i: (0, 0)),
            pl.BlockSpec((1, _DOUT), lambda i: (0, 0)),
        ],
        out_specs=[
            pl.BlockSpec((1, _N, _DOUT), lambda i: (i, 0, 0)),
            pl.BlockSpec((1, _N, 4), lambda i: (i, 0, 0)),
        ],
        out_shape=[
            jax.ShapeDtypeStruct((_BT, _N, _DOUT), jnp.float32),
            jax.ShapeDtypeStruct((_BT, _N, 4), jnp.float32),
        ],
        compiler_params=pltpu.CompilerParams(
            dimension_semantics=("parallel",),
        ),
    )(x, gs, wcat, bcat, att_W, att_a, att_h_row, fc_Wt, fc_b_row)

    y = y.reshape(_B, _T, _N, _DOUT)
    scores = s.reshape(_B, _T, _N, 4).transpose(0, 1, 3, 2)[..., None]
    return (y, scores)


# speedup vs baseline: 1.5725x; 1.5725x over previous
import jax
import jax.numpy as jnp
from jax.experimental import pallas as pl
from jax.experimental.pallas import tpu as pltpu

_B, _T, _N = 4, 8, 1024
_DIN, _DH, _DOUT = 256, 256, 256
_HID = _DH // 4
_BT = _B * _T
_BF = jnp.bfloat16


def _dhg_body(feats_ref, gs_ref, wcat_ref, bcat_ref, att_W_ref, att_a_ref,
              att_h_ref, fc_Wt_ref, fc_b_ref, out_ref, sc_ref):
    x = feats_ref[0].astype(_BF)
    h = jnp.dot(x, wcat_ref[...], preferred_element_type=jnp.float32)
    h = h.astype(_BF) + bcat_ref[...]

    a0 = att_a_ref[0:_HID, :]
    a1 = att_a_ref[_HID:2 * _HID, :]
    v2 = jnp.dot(att_W_ref[...], a0,
                 preferred_element_type=jnp.float32)
    hw = jnp.dot(att_h_ref[...], att_W_ref[...],
                 preferred_element_type=jnp.float32)
    c = jnp.dot(hw, a1, preferred_element_type=jnp.float32)

    branches = []
    for i in range(4):
        ai = jnp.dot(gs_ref[i], h[:, i * _DH:(i + 1) * _DH],
                     preferred_element_type=jnp.float32)
        branches.append(jnp.maximum(ai, 0.0).astype(_BF))
    acat = jnp.concatenate(branches, axis=1)

    v4 = jnp.concatenate([v2, v2, v2, v2], axis=0)
    row = jax.lax.broadcasted_iota(jnp.int32, (4 * _DH, 4), 0) // _DH
    col = jax.lax.broadcasted_iota(jnp.int32, (4 * _DH, 4), 1)
    vblk = jnp.where(row == col, v4, 0.0).astype(_BF)

    e = jnp.dot(acat, vblk, preferred_element_type=jnp.float32) + c
    e = jnp.where(e >= 0.0, e, 0.01 * e)
    m = jnp.max(e, axis=1, keepdims=True)
    ex = jnp.exp(e - m)
    s = ex / jnp.sum(ex, axis=1, keepdims=True)
    sc_ref[0] = s

    rowp = jax.lax.broadcasted_iota(jnp.int32, (4, 4 * _DH), 0)
    colp = jax.lax.broadcasted_iota(jnp.int32, (4, 4 * _DH), 1) // _DH
    pones = jnp.where(rowp == colp, 1.0, 0.0).astype(_BF)
    sf = jnp.dot(s.astype(_BF), pones,
                 preferred_element_type=jnp.float32).astype(_BF)

    w = sf[:, 0:_DH] * branches[0]
    for i in range(1, 4):
        w = w + sf[:, i * _DH:(i + 1) * _DH] * branches[i]
    w = jnp.maximum(w, jnp.bfloat16(0))

    y = jnp.dot(w, fc_Wt_ref[...],
                preferred_element_type=jnp.float32) + fc_b_ref[...]
    out_ref[0] = jnp.maximum(y, 0.0)


@jax.jit
def kernel(feats, G0, G1, G2, G3, W0, b0, W1, b1, W2, b2, W3, b3,
           att_W, att_h, att_a, fc_W, fc_b):
    gs = jnp.stack([G1, G0, G3, G2]).astype(_BF)
    wcat = jnp.concatenate([W1, W0, W3, W2], axis=1).astype(_BF)
    bcat = jnp.concatenate([b1, b0, b3, b2]).reshape(1, 4 * _DH).astype(_BF)
    x = feats.reshape(_BT, _N, _DIN)
    fc_Wt = fc_W.T.astype(_BF)
    att_h_row = att_h.reshape(1, _DH)
    fc_b_row = fc_b.reshape(1, _DOUT)

    y, s = pl.pallas_call(
        _dhg_body,
        grid=(_BT,),
        in_specs=[
            pl.BlockSpec((1, _N, _DIN), lambda i: (i, 0, 0)),
            pl.BlockSpec((4, _N, _N), lambda i: (0, 0, 0)),
            pl.BlockSpec((_DIN, 4 * _DH), lambda i: (0, 0)),
            pl.BlockSpec((1, 4 * _DH), lambda i: (0, 0)),
            pl.BlockSpec((_DH, _HID), lambda i: (0, 0)),
            pl.BlockSpec((2 * _HID, 1), lambda i: (0, 0)),
            pl.BlockSpec((1, _DH), lambda i: (0, 0)),
            pl.BlockSpec((_DH, _DOUT), lambda i: (0, 0)),
            pl.BlockSpec((1, _DOUT), lambda i: (0, 0)),
        ],
        out_specs=[
            pl.BlockSpec((1, _N, _DOUT), lambda i: (i, 0, 0)),
            pl.BlockSpec((1, _N, 4), lambda i: (i, 0, 0)),
        ],
        out_shape=[
            jax.ShapeDtypeStruct((_BT, _N, _DOUT), jnp.float32),
            jax.ShapeDtypeStruct((_BT, _N, 4), jnp.float32),
        ],
        compiler_params=pltpu.CompilerParams(
            dimension_semantics=("arbitrary",),
        ),
    )(x, gs, wcat, bcat, att_W, att_a, att_h_row, fc_Wt, fc_b_row)

    y = y.reshape(_B, _T, _N, _DOUT)
    scores = s.reshape(_B, _T, _N, 4).transpose(0, 1, 3, 2)[..., None]
    return (y, scores)


# speedup vs baseline: 1.5967x; 1.0154x over previous
import jax
import jax.numpy as jnp
from jax.experimental import pallas as pl
from jax.experimental.pallas import tpu as pltpu

_B, _T, _N = 4, 8, 1024
_DIN, _DH, _DOUT = 256, 256, 256
_HID = _DH // 4
_BT = _B * _T
_S = 1
_BF = jnp.bfloat16


def _dhg_body(feats_ref, gs_ref, wcat_ref, bcat_ref, att_W_ref, att_a_ref,
              att_h_ref, fc_Wt_ref, fc_b_ref, out_ref, sc_ref):
    a0 = att_a_ref[0:_HID, :]
    a1 = att_a_ref[_HID:2 * _HID, :]
    v2 = jnp.dot(att_W_ref[...], a0,
                 preferred_element_type=jnp.float32)
    hw = jnp.dot(att_h_ref[...], att_W_ref[...],
                 preferred_element_type=jnp.float32)
    c = jnp.dot(hw, a1, preferred_element_type=jnp.float32)

    v4 = jnp.concatenate([v2, v2, v2, v2], axis=0)
    row = jax.lax.broadcasted_iota(jnp.int32, (4 * _DH, 4), 0) // _DH
    col = jax.lax.broadcasted_iota(jnp.int32, (4 * _DH, 4), 1)
    vblk = jnp.where(row == col, v4, 0.0).astype(_BF)

    rowp = jax.lax.broadcasted_iota(jnp.int32, (4, 4 * _DH), 0)
    colp = jax.lax.broadcasted_iota(jnp.int32, (4, 4 * _DH), 1) // _DH
    pones = jnp.where(rowp == colp, 1.0, 0.0).astype(_BF)

    for s in range(_S):
        x = feats_ref[0, s].astype(_BF)
        h = jnp.dot(x, wcat_ref[...], preferred_element_type=jnp.float32)
        h = h.astype(_BF) + bcat_ref[...]

        branches = []
        for i in range(4):
            ai = jnp.dot(gs_ref[i], h[:, i * _DH:(i + 1) * _DH],
                         preferred_element_type=jnp.float32)
            branches.append(jnp.maximum(ai, 0.0).astype(_BF))
        acat = jnp.concatenate(branches, axis=1)

        e = jnp.dot(acat, vblk, preferred_element_type=jnp.float32) + c
        e = jnp.where(e >= 0.0, e, 0.01 * e)
        m = jnp.max(e, axis=1, keepdims=True)
        ex = jnp.exp(e - m)
        sm = ex / jnp.sum(ex, axis=1, keepdims=True)
        sc_ref[0, s] = sm

        sf = jnp.dot(sm.astype(_BF), pones,
                     preferred_element_type=jnp.float32).astype(_BF)

        w = sf[:, 0:_DH] * branches[0]
        for i in range(1, 4):
            w = w + sf[:, i * _DH:(i + 1) * _DH] * branches[i]
        w = jnp.maximum(w, jnp.bfloat16(0))

        y = jnp.dot(w, fc_Wt_ref[...],
                    preferred_element_type=jnp.float32) + fc_b_ref[...]
        out_ref[0, s] = jnp.maximum(y, 0.0)


@jax.jit
def kernel(feats, G0, G1, G2, G3, W0, b0, W1, b1, W2, b2, W3, b3,
           att_W, att_h, att_a, fc_W, fc_b):
    gs = jnp.stack([G1, G0, G3, G2]).astype(_BF)
    wcat = jnp.concatenate([W1, W0, W3, W2], axis=1).astype(_BF)
    bcat = jnp.concatenate([b1, b0, b3, b2]).reshape(1, 4 * _DH).astype(_BF)
    x = feats.reshape(_BT // _S, _S, _N, _DIN)
    fc_Wt = fc_W.T.astype(_BF)
    att_h_row = att_h.reshape(1, _DH)
    fc_b_row = fc_b.reshape(1, _DOUT)

    y, s = pl.pallas_call(
        _dhg_body,
        grid=(_BT // _S,),
        in_specs=[
            pl.BlockSpec((1, _S, _N, _DIN), lambda i: (i, 0, 0, 0)),
            pl.BlockSpec((4, _N, _N), lambda i: (0, 0, 0)),
            pl.BlockSpec((_DIN, 4 * _DH), lambda i: (0, 0)),
            pl.BlockSpec((1, 4 * _DH), lambda i: (0, 0)),
            pl.BlockSpec((_DH, _HID), lambda i: (0, 0)),
            pl.BlockSpec((2 * _HID, 1), lambda i: (0, 0)),
            pl.BlockSpec((1, _DH), lambda i: (0, 0)),
            pl.BlockSpec((_DH, _DOUT), lambda i: (0, 0)),
            pl.BlockSpec((1, _DOUT), lambda i: (0, 0)),
        ],
        out_specs=[
            pl.BlockSpec((1, _S, _N, _DOUT), lambda i: (i, 0, 0, 0)),
            pl.BlockSpec((1, _S, _N, 4), lambda i: (i, 0, 0, 0)),
        ],
        out_shape=[
            jax.ShapeDtypeStruct((_BT // _S, _S, _N, _DOUT), jnp.float32),
            jax.ShapeDtypeStruct((_BT // _S, _S, _N, 4), jnp.float32),
        ],
        compiler_params=pltpu.CompilerParams(
            dimension_semantics=("arbitrary",),
        ),
    )(x, gs, wcat, bcat, att_W, att_a, att_h_row, fc_Wt, fc_b_row)

    y = y.reshape(_B, _T, _N, _DOUT)
    scores = s.reshape(_B, _T, _N, 4).transpose(0, 1, 3, 2)[..., None]
    return (y, scores)


# speedup vs baseline: 1.7225x; 1.0787x over previous
import jax
import jax.numpy as jnp
from jax.experimental import pallas as pl
from jax.experimental.pallas import tpu as pltpu

_B, _T, _N = 4, 8, 1024
_DIN, _DH, _DOUT = 256, 256, 256
_HID = _DH // 4
_BT = _B * _T
_BF = jnp.bfloat16


def _dhg_body(feats_ref, g1_ref, g0_ref, g3_ref, g2_ref, wcat_ref, bcat_ref,
              att_W_ref, att_a_ref, att_h_ref, fc_Wt_ref, fc_b_ref,
              out_ref, sc_ref, gsb_ref):
    @pl.when(pl.program_id(0) == 0)
    def _cast_g():
        gsb_ref[0] = g1_ref[...].astype(_BF)
        gsb_ref[1] = g0_ref[...].astype(_BF)
        gsb_ref[2] = g3_ref[...].astype(_BF)
        gsb_ref[3] = g2_ref[...].astype(_BF)

    a0 = att_a_ref[0:_HID, :]
    a1 = att_a_ref[_HID:2 * _HID, :]
    v2 = jnp.dot(att_W_ref[...], a0,
                 preferred_element_type=jnp.float32)
    hw = jnp.dot(att_h_ref[...], att_W_ref[...],
                 preferred_element_type=jnp.float32)
    c = jnp.dot(hw, a1, preferred_element_type=jnp.float32)

    v4 = jnp.concatenate([v2, v2, v2, v2], axis=0)
    row = jax.lax.broadcasted_iota(jnp.int32, (4 * _DH, 4), 0) // _DH
    col = jax.lax.broadcasted_iota(jnp.int32, (4 * _DH, 4), 1)
    vblk = jnp.where(row == col, v4, 0.0).astype(_BF)

    rowp = jax.lax.broadcasted_iota(jnp.int32, (4, 4 * _DH), 0)
    colp = jax.lax.broadcasted_iota(jnp.int32, (4, 4 * _DH), 1) // _DH
    pones = jnp.where(rowp == colp, 1.0, 0.0).astype(_BF)

    x = feats_ref[0].astype(_BF)
    h = jnp.dot(x, wcat_ref[...], preferred_element_type=jnp.float32)
    h = h.astype(_BF) + bcat_ref[...]

    branches = []
    for i in range(4):
        ai = jnp.dot(gsb_ref[i], h[:, i * _DH:(i + 1) * _DH],
                     preferred_element_type=jnp.float32)
        branches.append(jnp.maximum(ai, 0.0).astype(_BF))
    acat = jnp.concatenate(branches, axis=1)

    e = jnp.dot(acat, vblk, preferred_element_type=jnp.float32) + c
    e = jnp.where(e >= 0.0, e, 0.01 * e)
    m = jnp.max(e, axis=1, keepdims=True)
    ex = jnp.exp(e - m)
    sm = ex / jnp.sum(ex, axis=1, keepdims=True)
    sc_ref[0] = sm.T

    sf = jnp.dot(sm.astype(_BF), pones,
                 preferred_element_type=jnp.float32).astype(_BF)

    w = sf[:, 0:_DH] * branches[0]
    for i in range(1, 4):
        w = w + sf[:, i * _DH:(i + 1) * _DH] * branches[i]
    w = jnp.maximum(w, jnp.bfloat16(0))

    y = jnp.dot(w, fc_Wt_ref[...],
                preferred_element_type=jnp.float32) + fc_b_ref[...]
    out_ref[0] = jnp.maximum(y, 0.0)


@jax.jit
def kernel(feats, G0, G1, G2, G3, W0, b0, W1, b1, W2, b2, W3, b3,
           att_W, att_h, att_a, fc_W, fc_b):
    wcat = jnp.concatenate([W1, W0, W3, W2], axis=1).astype(_BF)
    bcat = jnp.concatenate([b1, b0, b3, b2]).reshape(1, 4 * _DH).astype(_BF)
    x = feats.reshape(_BT, _N, _DIN)
    fc_Wt = fc_W.T.astype(_BF)
    att_h_row = att_h.reshape(1, _DH)
    fc_b_row = fc_b.reshape(1, _DOUT)

    gspec = pl.BlockSpec((_N, _N), lambda i: (0, 0))
    y, s = pl.pallas_call(
        _dhg_body,
        grid=(_BT,),
        in_specs=[
            pl.BlockSpec((1, _N, _DIN), lambda i: (i, 0, 0)),
            gspec, gspec, gspec, gspec,
            pl.BlockSpec((_DIN, 4 * _DH), lambda i: (0, 0)),
            pl.BlockSpec((1, 4 * _DH), lambda i: (0, 0)),
            pl.BlockSpec((_DH, _HID), lambda i: (0, 0)),
            pl.BlockSpec((2 * _HID, 1), lambda i: (0, 0)),
            pl.BlockSpec((1, _DH), lambda i: (0, 0)),
            pl.BlockSpec((_DH, _DOUT), lambda i: (0, 0)),
            pl.BlockSpec((1, _DOUT), lambda i: (0, 0)),
        ],
        out_specs=[
            pl.BlockSpec((1, _N, _DOUT), lambda i: (i, 0, 0)),
            pl.BlockSpec((1, 4, _N), lambda i: (i, 0, 0)),
        ],
        out_shape=[
            jax.ShapeDtypeStruct((_BT, _N, _DOUT), jnp.float32),
            jax.ShapeDtypeStruct((_BT, 4, _N), jnp.float32),
        ],
        scratch_shapes=[pltpu.VMEM((4, _N, _N), _BF)],
        compiler_params=pltpu.CompilerParams(
            dimension_semantics=("arbitrary",),
        ),
    )(x, G1, G0, G3, G2, wcat, bcat, att_W, att_a, att_h_row, fc_Wt, fc_b_row)

    y = y.reshape(_B, _T, _N, _DOUT)
    scores = s.reshape(_B, _T, 4, _N)[..., None]
    return (y, scores)


# speedup vs baseline: 1.8951x; 1.1002x over previous
import jax
import jax.numpy as jnp
from jax.experimental import pallas as pl
from jax.experimental.pallas import tpu as pltpu

_B, _T, _N = 4, 8, 1024
_DIN, _DH, _DOUT = 256, 256, 256
_HID = _DH // 4
_BT = _B * _T
_S = 2
_BF = jnp.bfloat16


def _dhg_body(feats_ref, g1_ref, g0_ref, g3_ref, g2_ref, wcat_ref, bcat_ref,
              att_W_ref, att_a_ref, att_h_ref, fc_Wt_ref, fc_b_ref,
              out_ref, sc_ref, gsb_ref):
    @pl.when(pl.program_id(0) == 0)
    def _cast_g():
        gsb_ref[0] = g1_ref[...].astype(_BF)
        gsb_ref[1] = g0_ref[...].astype(_BF)
        gsb_ref[2] = g3_ref[...].astype(_BF)
        gsb_ref[3] = g2_ref[...].astype(_BF)

    a0 = att_a_ref[0:_HID, :]
    a1 = att_a_ref[_HID:2 * _HID, :]
    v2 = jnp.dot(att_W_ref[...], a0,
                 preferred_element_type=jnp.float32)
    hw = jnp.dot(att_h_ref[...], att_W_ref[...],
                 preferred_element_type=jnp.float32)
    c = jnp.dot(hw, a1, preferred_element_type=jnp.float32)

    v4 = jnp.concatenate([v2, v2, v2, v2], axis=0)
    row = jax.lax.broadcasted_iota(jnp.int32, (4 * _DH, 4), 0) // _DH
    col = jax.lax.broadcasted_iota(jnp.int32, (4 * _DH, 4), 1)
    vblk = jnp.where(row == col, v4, 0.0).astype(_BF)

    rowp = jax.lax.broadcasted_iota(jnp.int32, (4, 4 * _DH), 0)
    colp = jax.lax.broadcasted_iota(jnp.int32, (4, 4 * _DH), 1) // _DH
    pones = jnp.where(rowp == colp, 1.0, 0.0).astype(_BF)

    def conv(s):
        x = feats_ref[0, s].astype(_BF)
        h = jnp.dot(x, wcat_ref[...], preferred_element_type=jnp.float32)
        h = h.astype(_BF) + bcat_ref[...]
        branches = []
        for i in range(4):
            ai = jnp.dot(gsb_ref[i], h[:, i * _DH:(i + 1) * _DH],
                         preferred_element_type=jnp.float32)
            branches.append(jnp.maximum(ai, 0.0).astype(_BF))
        return jnp.concatenate(branches, axis=1)

    def logits(acat):
        e = jnp.dot(acat, vblk, preferred_element_type=jnp.float32) + c
        e = jnp.where(e >= 0.0, e, 0.01 * e)
        m = jnp.max(e, axis=1, keepdims=True)
        ex = jnp.exp(e - m)
        return ex / jnp.sum(ex, axis=1, keepdims=True)

    def combine(s, sm, acat):
        sc_ref[0, s] = sm.T
        sf = jnp.dot(sm.astype(_BF), pones,
                     preferred_element_type=jnp.float32).astype(_BF)
        w = sf[:, 0:_DH] * acat[:, 0:_DH]
        for i in range(1, 4):
            w = w + sf[:, i * _DH:(i + 1) * _DH] * acat[:, i * _DH:(i + 1) * _DH]
        w = jnp.maximum(w, jnp.bfloat16(0))
        y = jnp.dot(w, fc_Wt_ref[...],
                    preferred_element_type=jnp.float32) + fc_b_ref[...]
        out_ref[0, s] = jnp.maximum(y, 0.0)

    acat0 = conv(0)

    x1 = feats_ref[0, 1].astype(_BF)
    h1 = jnp.dot(x1, wcat_ref[...], preferred_element_type=jnp.float32)
    h1 = h1.astype(_BF) + bcat_ref[...]

    sm0 = logits(acat0)

    b1 = []
    for i in range(4):
        ai = jnp.dot(gsb_ref[i], h1[:, i * _DH:(i + 1) * _DH],
                     preferred_element_type=jnp.float32)
        b1.append(jnp.maximum(ai, 0.0).astype(_BF))
        if i == 1:
            combine(0, sm0, acat0)
    acat1 = jnp.concatenate(b1, axis=1)

    sm1 = logits(acat1)
    combine(1, sm1, acat1)


@jax.jit
def kernel(feats, G0, G1, G2, G3, W0, b0, W1, b1, W2, b2, W3, b3,
           att_W, att_h, att_a, fc_W, fc_b):
    wcat = jnp.concatenate([W1, W0, W3, W2], axis=1).astype(_BF)
    bcat = jnp.concatenate([b1, b0, b3, b2]).reshape(1, 4 * _DH).astype(_BF)
    x = feats.reshape(_BT // _S, _S, _N, _DIN)
    fc_Wt = fc_W.T.astype(_BF)
    att_h_row = att_h.reshape(1, _DH)
    fc_b_row = fc_b.reshape(1, _DOUT)

    gspec = pl.BlockSpec((_N, _N), lambda i: (0, 0))
    y, s = pl.pallas_call(
        _dhg_body,
        grid=(_BT // _S,),
        in_specs=[
            pl.BlockSpec((1, _S, _N, _DIN), lambda i: (i, 0, 0, 0)),
            gspec, gspec, gspec, gspec,
            pl.BlockSpec((_DIN, 4 * _DH), lambda i: (0, 0)),
            pl.BlockSpec((1, 4 * _DH), lambda i: (0, 0)),
            pl.BlockSpec((_DH, _HID), lambda i: (0, 0)),
            pl.BlockSpec((2 * _HID, 1), lambda i: (0, 0)),
            pl.BlockSpec((1, _DH), lambda i: (0, 0)),
            pl.BlockSpec((_DH, _DOUT), lambda i: (0, 0)),
            pl.BlockSpec((1, _DOUT), lambda i: (0, 0)),
        ],
        out_specs=[
            pl.BlockSpec((1, _S, _N, _DOUT), lambda i: (i, 0, 0, 0)),
            pl.BlockSpec((1, _S, 4, _N), lambda i: (i, 0, 0, 0)),
        ],
        out_shape=[
            jax.ShapeDtypeStruct((_BT // _S, _S, _N, _DOUT), jnp.float32),
            jax.ShapeDtypeStruct((_BT // _S, _S, 4, _N), jnp.float32),
        ],
        scratch_shapes=[pltpu.VMEM((4, _N, _N), _BF)],
        compiler_params=pltpu.CompilerParams(
            dimension_semantics=("arbitrary",),
        ),
    )(x, G1, G0, G3, G2, wcat, bcat, att_W, att_a, att_h_row, fc_Wt, fc_b_row)

    y = y.reshape(_B, _T, _N, _DOUT)
    scores = s.reshape(_B, _T, 4, _N)[..., None]
    return (y, scores)


# speedup vs baseline: 2.0234x; 1.0677x over previous
import jax
import jax.numpy as jnp
from jax.experimental import pallas as pl
from jax.experimental.pallas import tpu as pltpu

_B, _T, _N = 4, 8, 1024
_DIN, _DH, _DOUT = 256, 256, 256
_HID = _DH // 4
_BT = _B * _T
_S = 4
_BF = jnp.bfloat16


def _dhg_body(feats_ref, g1_ref, g0_ref, g3_ref, g2_ref, wcat_ref, bcat_ref,
              att_W_ref, att_a_ref, att_h_ref, fc_Wt_ref, fc_b_ref,
              out_ref, sc_ref, gsb_ref):
    @pl.when(pl.program_id(0) == 0)
    def _cast_g():
        gsb_ref[0] = g1_ref[...].astype(_BF)
        gsb_ref[1] = g0_ref[...].astype(_BF)
        gsb_ref[2] = g3_ref[...].astype(_BF)
        gsb_ref[3] = g2_ref[...].astype(_BF)

    a0 = att_a_ref[0:_HID, :]
    a1 = att_a_ref[_HID:2 * _HID, :]
    v2 = jnp.dot(att_W_ref[...], a0,
                 preferred_element_type=jnp.float32)
    hw = jnp.dot(att_h_ref[...], att_W_ref[...],
                 preferred_element_type=jnp.float32)
    c = jnp.dot(hw, a1, preferred_element_type=jnp.float32)

    v4 = jnp.concatenate([v2, v2, v2, v2], axis=0)
    row = jax.lax.broadcasted_iota(jnp.int32, (4 * _DH, 4), 0) // _DH
    col = jax.lax.broadcasted_iota(jnp.int32, (4 * _DH, 4), 1)
    vblk = jnp.where(row == col, v4, 0.0).astype(_BF)

    rowp = jax.lax.broadcasted_iota(jnp.int32, (4, 4 * _DH), 0)
    colp = jax.lax.broadcasted_iota(jnp.int32, (4, 4 * _DH), 1) // _DH
    pones = jnp.where(rowp == colp, 1.0, 0.0).astype(_BF)

    def conv(s):
        x = feats_ref[0, s].astype(_BF)
        h = jnp.dot(x, wcat_ref[...], preferred_element_type=jnp.float32)
        h = h.astype(_BF) + bcat_ref[...]
        branches = []
        for i in range(4):
            ai = jnp.dot(gsb_ref[i], h[:, i * _DH:(i + 1) * _DH],
                         preferred_element_type=jnp.float32)
            branches.append(jnp.maximum(ai, 0.0).astype(_BF))
        return jnp.concatenate(branches, axis=1)

    def logits(acat):
        e = jnp.dot(acat, vblk, preferred_element_type=jnp.float32) + c
        e = jnp.where(e >= 0.0, e, 0.01 * e)
        m = jnp.max(e, axis=1, keepdims=True)
        ex = jnp.exp(e - m)
        return ex / jnp.sum(ex, axis=1, keepdims=True)

    def combine(s, sm, acat):
        sc_ref[0, s] = sm.T
        sf = jnp.dot(sm.astype(_BF), pones,
                     preferred_element_type=jnp.float32).astype(_BF)
        w = sf[:, 0:_DH] * acat[:, 0:_DH]
        for i in range(1, 4):
            w = w + sf[:, i * _DH:(i + 1) * _DH] * acat[:, i * _DH:(i + 1) * _DH]
        w = jnp.maximum(w, jnp.bfloat16(0))
        y = jnp.dot(w, fc_Wt_ref[...],
                    preferred_element_type=jnp.float32) + fc_b_ref[...]
        out_ref[0, s] = jnp.maximum(y, 0.0)

    acat = conv(0)
    for s in range(1, _S):
        xs = feats_ref[0, s].astype(_BF)
        hs = jnp.dot(xs, wcat_ref[...], preferred_element_type=jnp.float32)
        hs = hs.astype(_BF) + bcat_ref[...]
        sm_prev = logits(acat)
        bs = []
        for i in range(4):
            ai = jnp.dot(gsb_ref[i], hs[:, i * _DH:(i + 1) * _DH],
                         preferred_element_type=jnp.float32)
            bs.append(jnp.maximum(ai, 0.0).astype(_BF))
            if i == 1:
                combine(s - 1, sm_prev, acat)
        acat = jnp.concatenate(bs, axis=1)

    combine(_S - 1, logits(acat), acat)


@jax.jit
def kernel(feats, G0, G1, G2, G3, W0, b0, W1, b1, W2, b2, W3, b3,
           att_W, att_h, att_a, fc_W, fc_b):
    wcat = jnp.concatenate([W1, W0, W3, W2], axis=1).astype(_BF)
    bcat = jnp.concatenate([b1, b0, b3, b2]).reshape(1, 4 * _DH).astype(_BF)
    x = feats.reshape(_BT // _S, _S, _N, _DIN)
    fc_Wt = fc_W.T.astype(_BF)
    att_h_row = att_h.reshape(1, _DH)
    fc_b_row = fc_b.reshape(1, _DOUT)

    gspec = pl.BlockSpec((_N, _N), lambda i: (0, 0))
    y, s = pl.pallas_call(
        _dhg_body,
        grid=(_BT // _S,),
        in_specs=[
            pl.BlockSpec((1, _S, _N, _DIN), lambda i: (i, 0, 0, 0)),
            gspec, gspec, gspec, gspec,
            pl.BlockSpec((_DIN, 4 * _DH), lambda i: (0, 0)),
            pl.BlockSpec((1, 4 * _DH), lambda i: (0, 0)),
            pl.BlockSpec((_DH, _HID), lambda i: (0, 0)),
            pl.BlockSpec((2 * _HID, 1), lambda i: (0, 0)),
            pl.BlockSpec((1, _DH), lambda i: (0, 0)),
            pl.BlockSpec((_DH, _DOUT), lambda i: (0, 0)),
            pl.BlockSpec((1, _DOUT), lambda i: (0, 0)),
        ],
        out_specs=[
            pl.BlockSpec((1, _S, _N, _DOUT), lambda i: (i, 0, 0, 0)),
            pl.BlockSpec((1, _S, 4, _N), lambda i: (i, 0, 0, 0)),
        ],
        out_shape=[
            jax.ShapeDtypeStruct((_BT // _S, _S, _N, _DOUT), jnp.float32),
            jax.ShapeDtypeStruct((_BT // _S, _S, 4, _N), jnp.float32),
        ],
        scratch_shapes=[pltpu.VMEM((4, _N, _N), _BF)],
        compiler_params=pltpu.CompilerParams(
            dimension_semantics=("arbitrary",),
        ),
    )(x, G1, G0, G3, G2, wcat, bcat, att_W, att_a, att_h_row, fc_Wt, fc_b_row)

    y = y.reshape(_B, _T, _N, _DOUT)
    scores = s.reshape(_B, _T, 4, _N)[..., None]
    return (y, scores)


# speedup vs baseline: 2.1094x; 1.0425x over previous
import jax
import jax.numpy as jnp
from jax.experimental import pallas as pl
from jax.experimental.pallas import tpu as pltpu

_B, _T, _N = 4, 8, 1024
_DIN, _DH, _DOUT = 256, 256, 256
_HID = _DH // 4
_BT = _B * _T
_S = 4
_BF = jnp.bfloat16


def _dhg_body(feats_ref, g1_ref, g0_ref, g3_ref, g2_ref, wcat_ref, bcat_ref,
              att_W_ref, att_a_ref, att_h_ref, fc_Wt_ref, fc_b_ref,
              out_ref, sc_ref, gsb_ref):
    @pl.when(pl.program_id(0) == 0)
    def _cast_g():
        gsb_ref[0] = g1_ref[...].astype(_BF)
        gsb_ref[1] = g0_ref[...].astype(_BF)
        gsb_ref[2] = g3_ref[...].astype(_BF)
        gsb_ref[3] = g2_ref[...].astype(_BF)

    a0 = att_a_ref[0:_HID, :]
    a1 = att_a_ref[_HID:2 * _HID, :]
    v2 = jnp.dot(att_W_ref[...], a0,
                 preferred_element_type=jnp.float32)
    hw = jnp.dot(att_h_ref[...], att_W_ref[...],
                 preferred_element_type=jnp.float32)
    c = jnp.dot(hw, a1, preferred_element_type=jnp.float32)

    v4 = jnp.concatenate([v2, v2, v2, v2], axis=0)
    row = jax.lax.broadcasted_iota(jnp.int32, (4 * _DH, 4), 0) // _DH
    col = jax.lax.broadcasted_iota(jnp.int32, (4 * _DH, 4), 1)
    vblk = jnp.where(row == col, v4, 0.0).astype(_BF)

    rowp = jax.lax.broadcasted_iota(jnp.int32, (4, 4 * _DH), 0)
    colp = jax.lax.broadcasted_iota(jnp.int32, (4, 4 * _DH), 1) // _DH
    pones = jnp.where(rowp == colp, 1.0, 0.0).astype(_BF)

    def conv(s):
        x = feats_ref[0, s].astype(_BF)
        h = jnp.dot(x, wcat_ref[...], preferred_element_type=jnp.float32)
        h = h.astype(_BF) + bcat_ref[...]
        branches = []
        for i in range(4):
            ai = jnp.dot(gsb_ref[i], h[:, i * _DH:(i + 1) * _DH],
                         preferred_element_type=jnp.float32)
            branches.append(jnp.maximum(ai, 0.0).astype(_BF))
        return jnp.concatenate(branches, axis=1)

    def logits(acat):
        e = jnp.dot(acat, vblk, preferred_element_type=jnp.float32) + c
        e = jnp.where(e >= 0.0, e, 0.01 * e)
        m = jnp.max(e, axis=1, keepdims=True)
        ex = jnp.exp(e - m)
        return ex / jnp.sum(ex, axis=1, keepdims=True)

    def combine(s, sm, acat):
        sc_ref[0, s] = sm.T
        sf = jnp.dot(sm.astype(_BF), pones,
                     preferred_element_type=jnp.float32).astype(_BF)
        w = sf[:, 0:_DH] * acat[:, 0:_DH]
        for i in range(1, 4):
            w = w + sf[:, i * _DH:(i + 1) * _DH] * acat[:, i * _DH:(i + 1) * _DH]
        w = jnp.maximum(w, jnp.bfloat16(0))
        y = jnp.dot(w, fc_Wt_ref[...],
                    preferred_element_type=jnp.float32) + fc_b_ref[...]
        out_ref[0, s] = jnp.maximum(y, 0.0)

    def make_h(s):
        xs = feats_ref[0, s].astype(_BF)
        hs = jnp.dot(xs, wcat_ref[...], preferred_element_type=jnp.float32)
        return hs.astype(_BF) + bcat_ref[...]

    h_cur = make_h(0)
    acat_prev = None
    sm_prev = None
    for s in range(_S):
        if s > 0:
            sm_prev = logits(acat_prev)
        bs = []
        h_next = None
        for i in range(4):
            ai = jnp.dot(gsb_ref[i], h_cur[:, i * _DH:(i + 1) * _DH],
                         preferred_element_type=jnp.float32)
            bs.append(jnp.maximum(ai, 0.0).astype(_BF))
            if i == 0 and s + 1 < _S:
                h_next = make_h(s + 1)
            if i == 2 and s > 0:
                combine(s - 1, sm_prev, acat_prev)
        acat_prev = jnp.concatenate(bs, axis=1)
        h_cur = h_next

    combine(_S - 1, logits(acat_prev), acat_prev)


@jax.jit
def kernel(feats, G0, G1, G2, G3, W0, b0, W1, b1, W2, b2, W3, b3,
           att_W, att_h, att_a, fc_W, fc_b):
    wcat = jnp.concatenate([W1, W0, W3, W2], axis=1).astype(_BF)
    bcat = jnp.concatenate([b1, b0, b3, b2]).reshape(1, 4 * _DH).astype(_BF)
    x = feats.reshape(_BT // _S, _S, _N, _DIN)
    fc_Wt = fc_W.T.astype(_BF)
    att_h_row = att_h.reshape(1, _DH)
    fc_b_row = fc_b.reshape(1, _DOUT)

    gspec = pl.BlockSpec((_N, _N), lambda i: (0, 0))
    y, s = pl.pallas_call(
        _dhg_body,
        grid=(_BT // _S,),
        in_specs=[
            pl.BlockSpec((1, _S, _N, _DIN), lambda i: (i, 0, 0, 0)),
            gspec, gspec, gspec, gspec,
            pl.BlockSpec((_DIN, 4 * _DH), lambda i: (0, 0)),
            pl.BlockSpec((1, 4 * _DH), lambda i: (0, 0)),
            pl.BlockSpec((_DH, _HID), lambda i: (0, 0)),
            pl.BlockSpec((2 * _HID, 1), lambda i: (0, 0)),
            pl.BlockSpec((1, _DH), lambda i: (0, 0)),
            pl.BlockSpec((_DH, _DOUT), lambda i: (0, 0)),
            pl.BlockSpec((1, _DOUT), lambda i: (0, 0)),
        ],
        out_specs=[
            pl.BlockSpec((1, _S, _N, _DOUT), lambda i: (i, 0, 0, 0)),
            pl.BlockSpec((1, _S, 4, _N), lambda i: (i, 0, 0, 0)),
        ],
        out_shape=[
            jax.ShapeDtypeStruct((_BT // _S, _S, _N, _DOUT), jnp.float32),
            jax.ShapeDtypeStruct((_BT // _S, _S, 4, _N), jnp.float32),
        ],
        scratch_shapes=[pltpu.VMEM((4, _N, _N), _BF)],
        compiler_params=pltpu.CompilerParams(
            dimension_semantics=("arbitrary",),
        ),
    )(x, G1, G0, G3, G2, wcat, bcat, att_W, att_a, att_h_row, fc_Wt, fc_b_row)

    y = y.reshape(_B, _T, _N, _DOUT)
    scores = s.reshape(_B, _T, 4, _N)[..., None]
    return (y, scores)
